# baseline (speedup 1.0000x reference)
import functools
import math

import jax
import jax.numpy as jnp
from jax import lax
from jax.experimental import pallas as pl
from jax.experimental.pallas import tpu as pltpu
from jax.experimental.pallas import tpu_sc as plsc

N_BLOCK = 2
WIN = 256
D = 384
NH = 12
DH = D // NH
HID = int(D * 4.0)
B = 2
N = 2048
R = B * N
NWIN = R // WIN
EPS = 1e-5

SC_NC = 2
SC_NS = 16
SC_NW = SC_NC * SC_NS
ROWS_PER_W = R // SC_NW


_NCHK = 4
_CR = ROWS_PER_W // _NCHK


def _sc_scatter_body(table_hbm, idx_hbm, idx2_hbm, nidx_hbm,
                     out_hbm, pa2_hbm,
                     idx_v, idx2_v, nidx_v, rows_v, sem_r, sem_i):
    wid = lax.axis_index("s") * SC_NC + lax.axis_index("c")
    base = wid * ROWS_PER_W
    pltpu.sync_copy(idx_hbm.at[wid], idx_v)
    pltpu.sync_copy(idx2_hbm.at[wid], idx2_v)
    pltpu.sync_copy(nidx_hbm.at[wid], nidx_v)
    pltpu.sync_copy(table_hbm.at[pl.ds(base, ROWS_PER_W)], rows_v)
    c1 = pltpu.async_copy(rows_v, out_hbm.at[idx_v], sem_r)
    c2 = pltpu.async_copy(nidx_v, pa2_hbm.at[idx2_v], sem_i)
    c1.wait()
    c2.wait()


def _sc_gather_body(table_hbm, idx_hbm, out_hbm, idx_v, rows_v, sem_r):
    wid = lax.axis_index("s") * SC_NC + lax.axis_index("c")
    base = wid * ROWS_PER_W
    pltpu.sync_copy(idx_hbm.at[wid], idx_v)
    pltpu.async_copy(table_hbm.at[idx_v], rows_v, sem_r).wait()
    pltpu.sync_copy(rows_v, out_hbm.at[pl.ds(base, ROWS_PER_W)])


def _sc_mid_body(table_hbm, rank1_hbm, pa2_hbm, out_hbm,
                 pa2_v, g2_v, rows_v, sem_i, sem_r):
    wid = lax.axis_index("s") * SC_NC + lax.axis_index("c")
    base = wid * ROWS_PER_W
    pltpu.sync_copy(pa2_hbm.at[pl.ds(base, ROWS_PER_W)], pa2_v)
    pltpu.async_copy(rank1_hbm.at[pa2_v], g2_v, sem_i).wait()
    pltpu.async_copy(table_hbm.at[g2_v], rows_v, sem_r).wait()
    pltpu.sync_copy(rows_v, out_hbm.at[pl.ds(base, ROWS_PER_W)])


def _sc_mesh():
    return plsc.VectorSubcoreMesh(
        core_axis_name="c", subcore_axis_name="s",
        num_cores=SC_NC, num_subcores=SC_NS)


_ROWS_SCRATCH = pltpu.VMEM((ROWS_PER_W, D), jnp.float32)
_IDX_1D = pltpu.VMEM((ROWS_PER_W,), jnp.int32)
_ROWS_OUT = jax.ShapeDtypeStruct((R, D), jnp.float32)


@functools.cache
def _sc_move_kernel(kind):
    body, out_type, scratch = {
        "scatter": (_sc_scatter_body,
                    [_ROWS_OUT, jax.ShapeDtypeStruct((R,), jnp.int32)],
                    [_IDX_1D, _IDX_1D, _IDX_1D, _ROWS_SCRATCH,
                     pltpu.SemaphoreType.DMA, pltpu.SemaphoreType.DMA]),
        "gather": (_sc_gather_body, _ROWS_OUT,
                   [_IDX_1D, _ROWS_SCRATCH, pltpu.SemaphoreType.DMA]),
        "mid": (_sc_mid_body, _ROWS_OUT,
                [_IDX_1D, _IDX_1D, _ROWS_SCRATCH,
                 pltpu.SemaphoreType.DMA, pltpu.SemaphoreType.DMA]),
    }[kind]
    return pl.kernel(
        body,
        out_type=out_type,
        mesh=_sc_mesh(),
        scratch_types=scratch,
    )


def _sc_scatter(table, idx, idx2, nidx):
    return _sc_move_kernel("scatter")(table, idx, idx2, nidx)


def _sc_gather(table, idx):
    return _sc_move_kernel("gather")(table, idx)


def _sc_mid(table, rank1_flat, pa2):
    return _sc_move_kernel("mid")(table, rank1_flat, pa2)


def _bn_affine(st, gb_ref, grow, brow):
    mean = st[0:1, :] * (1.0 / R)
    var = st[1:2, :] * (1.0 / R) - mean * mean
    scale = gb_ref[grow:grow + 1, :] * lax.rsqrt(var + EPS)
    shift = gb_ref[brow:brow + 1, :] - mean * scale
    return scale, shift


def _out_stats(y, i, ost_ref):
    s = jnp.sum(y, axis=0, keepdims=True)
    ss = jnp.sum(y * y, axis=0, keepdims=True)
    blk = jnp.concatenate([s, ss, jnp.zeros((6, D), jnp.float32)], axis=0)

    @pl.when(i == 0)
    def _():
        ost_ref[...] = blk

    @pl.when(i > 0)
    def _():
        ost_ref[...] += blk


_HG = 4
_NG = NH // _HG
_GW = _HG * DH


def _nt(a, b, out_dtype=jnp.float32):
    return lax.dot_general(a, b, (((1,), (1,)), ((), ())),
                           preferred_element_type=out_dtype)


def _block_body(emit_stats, st_ref, gb_ref, x_ref, wqkv_ref, wproj_ref,
                w1_ref, w2_ref, y_ref, ost_ref, h_vmem, st2_vmem,
                wqkv_bf, wproj_bf, w1_bf, w2_bf):
    ph = pl.program_id(0)
    i = pl.program_id(1)

    @pl.when((ph == 0) & (i == 0))
    def _cast_weights():
        wq = wqkv_ref[0]
        rowid = lax.broadcasted_iota(jnp.int32, (3 * D, D), 0)
        wq = jnp.where(rowid < D, wq * (1.0 / math.sqrt(DH)), wq)
        wqkv_bf[...] = wq.astype(jnp.bfloat16)
        wproj_bf[...] = wproj_ref[0].astype(jnp.bfloat16)
        w1_bf[...] = w1_ref[0].astype(jnp.bfloat16)
        w2_bf[...] = w2_ref[0].astype(jnp.bfloat16)

    @pl.when(ph == 0)
    def _attn_phase():
        x = x_ref[...]
        scale, shift = _bn_affine(st_ref[...], gb_ref, 0, 1)
        xn = (x * scale + shift).astype(jnp.bfloat16)
        qkvb = _nt(xn, wqkv_bf[...]).astype(jnp.bfloat16)
        lane = lax.broadcasted_iota(jnp.int32, (1, _GW), 1)
        outs = []
        for g in range(_NG):
            q4 = qkvb[:, g * _GW:(g + 1) * _GW]
            k4 = qkvb[:, D + g * _GW:D + (g + 1) * _GW]
            v4 = qkvb[:, 2 * D + g * _GW:2 * D + (g + 1) * _GW]
            bdk = jnp.concatenate(
                [jnp.where((lane >= h * DH) & (lane < (h + 1) * DH), k4, 0)
                 for h in range(_HG)], axis=0)
            s4 = _nt(q4, bdk)
            e4 = jnp.exp(s4)
            p4 = jnp.concatenate(
                [e4[:, h * WIN:(h + 1) * WIN]
                 / jnp.sum(e4[:, h * WIN:(h + 1) * WIN], axis=-1,
                           keepdims=True) for h in range(_HG)],
                axis=1).astype(jnp.bfloat16)
            bdv = jnp.concatenate(
                [jnp.where((lane >= h * DH) & (lane < (h + 1) * DH), v4, 0)
                 for h in range(_HG)], axis=0)
            outs.append(jnp.dot(p4, bdv, preferred_element_type=jnp.float32))
        o = jnp.concatenate(outs, axis=1).astype(jnp.bfloat16)
        h_out = x + _nt(o, wproj_bf[...])
        h_vmem[pl.ds(i * WIN, WIN), :] = h_out
        s = jnp.sum(h_out, axis=0, keepdims=True)
        ss = jnp.sum(h_out * h_out, axis=0, keepdims=True)
        blk = jnp.concatenate([s, ss, jnp.zeros((6, D), jnp.float32)], axis=0)

        @pl.when(i == 0)
        def _():
            st2_vmem[...] = blk

        @pl.when(i > 0)
        def _():
            st2_vmem[...] += blk

    @pl.when(ph == 1)
    def _mlp_phase():
        hrow = h_vmem[pl.ds(i * WIN, WIN), :]
        scale, shift = _bn_affine(st2_vmem[...], gb_ref, 2, 3)
        hn = (hrow * scale + shift).astype(jnp.bfloat16)
        a = _nt(hn, w1_bf[...])
        a = jnp.maximum(a, 0.0).astype(jnp.bfloat16)
        y = hrow + _nt(a, w2_bf[...])
        y_ref[...] = y
        if emit_stats:
            _out_stats(y, i, ost_ref)


def _block_call(blk_i, st, gb, xp, qkv_w, proj_w, fc1_w, fc2_w,
                emit_stats=True):
    return pl.pallas_call(
        functools.partial(_block_body, emit_stats),
        grid=(2, NWIN),
        in_specs=[
            pl.BlockSpec((8, D), lambda p, i: (0, 0)),
            pl.BlockSpec((8, D), lambda p, i: (0, 0)),
            pl.BlockSpec((WIN, D), lambda p, i: (i * (1 - p), 0)),
            pl.BlockSpec((1, 3 * D, D), lambda p, i, b=blk_i: (b, 0, 0)),
            pl.BlockSpec((1, D, D), lambda p, i, b=blk_i: (b, 0, 0)),
            pl.BlockSpec((1, HID, D), lambda p, i, b=blk_i: (b, 0, 0)),
            pl.BlockSpec((1, D, HID), lambda p, i, b=blk_i: (b, 0, 0)),
        ],
        out_specs=[
            pl.BlockSpec((WIN, D), lambda p, i: (i, 0)),
            pl.BlockSpec((8, D), lambda p, i: (0, 0)),
        ],
        out_shape=[
            jax.ShapeDtypeStruct((R, D), jnp.float32),
            jax.ShapeDtypeStruct((8, D), jnp.float32),
        ],
        scratch_shapes=[
            pltpu.VMEM((R, D), jnp.float32),
            pltpu.VMEM((8, D), jnp.float32),
            pltpu.VMEM((3 * D, D), jnp.bfloat16),
            pltpu.VMEM((D, D), jnp.bfloat16),
            pltpu.VMEM((HID, D), jnp.bfloat16),
            pltpu.VMEM((D, HID), jnp.bfloat16),
        ],
    )(st, gb, xp, qkv_w, proj_w, fc1_w, fc2_w)


_CH = 256
_NCH = N // _CH


_XCH = R // (2 * B)


def _rank_body(prow_ref, x_ref, out_ref, st_ref):
    r = pl.program_id(0)
    xc = x_ref[...]
    s = jnp.sum(xc, axis=0, keepdims=True)
    ss = jnp.sum(xc * xc, axis=0, keepdims=True)
    blk = jnp.concatenate([s, ss, jnp.zeros((6, D), jnp.float32)], axis=0)

    @pl.when(r == 0)
    def _():
        st_ref[...] = blk

    @pl.when(r > 0)
    def _():
        st_ref[...] += blk

    prow = prow_ref[0]
    pr8 = jnp.concatenate(
        [prow[:, c * _CH:(c + 1) * _CH] for c in range(_NCH)], axis=0)
    tcol = jnp.transpose(pr8)
    tri = (lax.broadcasted_iota(jnp.int32, (_CH, _CH), 0)
           < lax.broadcasted_iota(jnp.int32, (_CH, _CH), 1))
    chunks = []
    for ci in range(_NCH):
        pi = prow[:, ci * _CH:(ci + 1) * _CH]
        acc = jnp.zeros((1, _CH), jnp.float32)
        for cj in range(_NCH):
            pj = tcol[:, cj:cj + 1]
            if cj < ci:
                cmp = pj <= pi
            elif cj > ci:
                cmp = pj < pi
            else:
                cmp = (pj < pi) | ((pj == pi) & tri)
            acc = acc + jnp.sum(cmp.astype(jnp.float32), axis=0, keepdims=True)
        chunks.append(acc)
    rank = jnp.concatenate(chunks, axis=1).astype(jnp.int32)
    out_ref[0] = rank + (r % 2) * N


def _rank_call(prow, xf):
    return pl.pallas_call(
        _rank_body,
        grid=(2 * B,),
        in_specs=[
            pl.BlockSpec((1, 1, N), lambda r: (r, 0, 0)),
            pl.BlockSpec((_XCH, D), lambda r: (r, 0)),
        ],
        out_specs=[
            pl.BlockSpec((1, 1, N), lambda r: (r, 0, 0)),
            pl.BlockSpec((8, D), lambda r: (0, 0)),
        ],
        out_shape=[
            jax.ShapeDtypeStruct((2 * B, 1, N), jnp.int32),
            jax.ShapeDtypeStruct((8, D), jnp.float32),
        ],
    )(prow, xf)


def _perm_indices(z, xf):
    kidx = jax.random.key(42)

    def get_proj(key):
        v = jax.random.normal(key, (3,), dtype=jnp.float32)
        v = v / jnp.linalg.norm(v)
        return jnp.einsum('bnc,c->bn', z, v)

    p1 = get_proj(jax.random.fold_in(kidx, 1))
    p2 = get_proj(jax.random.fold_in(kidx, 2))
    P = jnp.concatenate([p1, p2], axis=0)
    ranks3, st = _rank_call(P[:, None, :], xf)
    ranks = ranks3.reshape(2 * B, N)
    rank1_flat = ranks[0:B].reshape(R)
    idx1 = rank1_flat.reshape(SC_NW, ROWS_PER_W)
    idx2 = ranks[B:2 * B].reshape(SC_NW, ROWS_PER_W)
    return idx1, idx2, rank1_flat, st


def kernel(x, z, qkv_w, proj_w, fc1_w, fc2_w, bn1_g, bn1_b, bn2_g, bn2_b):
    xf = x.reshape(R, D)
    idx1, idx2, rank1_flat, st = _perm_indices(z, xf)
    nidx = jnp.arange(R, dtype=jnp.int32).reshape(SC_NW, ROWS_PER_W)

    zpad = jnp.zeros((4, D), jnp.float32)
    gb = [jnp.concatenate([bn1_g[i][None], bn1_b[i][None],
                           bn2_g[i][None], bn2_b[i][None], zpad], axis=0)
          for i in range(N_BLOCK)]

    xp, pa2 = _sc_scatter(xf, idx1, idx2, nidx)
    y, st3 = _block_call(0, st, gb[0], xp, qkv_w, proj_w, fc1_w, fc2_w)
    xp2 = _sc_mid(y, rank1_flat, pa2)
    y2, _ = _block_call(1, st3, gb[1], xp2, qkv_w, proj_w, fc1_w, fc2_w,
                        emit_stats=False)
    out = _sc_gather(y2, idx2)
    return out.reshape(B, N, D), z


# speedup vs baseline: 1.1228x; 1.1228x over previous
import functools
import math

import jax
import jax.numpy as jnp
from jax import lax
from jax.experimental import pallas as pl
from jax.experimental.pallas import tpu as pltpu
from jax.experimental.pallas import tpu_sc as plsc

N_BLOCK = 2
WIN = 256
D = 384
NH = 12
DH = D // NH
HID = int(D * 4.0)
B = 2
N = 2048
R = B * N
NWIN = R // WIN
EPS = 1e-5

SC_NC = 2
SC_NS = 16
SC_NW = SC_NC * SC_NS
ROWS_PER_W = R // SC_NW


_NCHK = 4
_CR = ROWS_PER_W // _NCHK


def _sc_scatter_body(table_hbm, idx_hbm, out_hbm, idx_v, rows_v, sem_r):
    wid = lax.axis_index("s") * SC_NC + lax.axis_index("c")
    base = wid * ROWS_PER_W
    pltpu.sync_copy(idx_hbm.at[wid], idx_v)
    pltpu.sync_copy(table_hbm.at[pl.ds(base, ROWS_PER_W)], rows_v)
    pltpu.async_copy(rows_v, out_hbm.at[idx_v], sem_r).wait()


def _sc_gather_body(table_hbm, idx_hbm, out_hbm, idx_v, rows_v, sem_r):
    wid = lax.axis_index("s") * SC_NC + lax.axis_index("c")
    base = wid * ROWS_PER_W
    pltpu.sync_copy(idx_hbm.at[wid], idx_v)
    pltpu.async_copy(table_hbm.at[idx_v], rows_v, sem_r).wait()
    pltpu.sync_copy(rows_v, out_hbm.at[pl.ds(base, ROWS_PER_W)])


def _sc_gs_body(table_hbm, idxg_hbm, idxs_hbm, out_hbm,
                idxg_v, idxs_v, rows_v, sem_g, sem_s):
    wid = lax.axis_index("s") * SC_NC + lax.axis_index("c")
    pltpu.sync_copy(idxg_hbm.at[wid], idxg_v)
    pltpu.sync_copy(idxs_hbm.at[wid], idxs_v)
    pltpu.async_copy(table_hbm.at[idxg_v], rows_v, sem_g).wait()
    pltpu.async_copy(rows_v, out_hbm.at[idxs_v], sem_s).wait()


def _sc_mesh():
    return plsc.VectorSubcoreMesh(
        core_axis_name="c", subcore_axis_name="s",
        num_cores=SC_NC, num_subcores=SC_NS)


_ROWS_SCRATCH = pltpu.VMEM((ROWS_PER_W, D), jnp.float32)
_IDX_1D = pltpu.VMEM((ROWS_PER_W,), jnp.int32)
_ROWS_OUT = jax.ShapeDtypeStruct((R, D), jnp.float32)


@functools.cache
def _sc_move_kernel(kind):
    body, scratch = {
        "scatter": (_sc_scatter_body,
                    [_IDX_1D, _ROWS_SCRATCH, pltpu.SemaphoreType.DMA]),
        "gather": (_sc_gather_body,
                   [_IDX_1D, _ROWS_SCRATCH, pltpu.SemaphoreType.DMA]),
        "gs": (_sc_gs_body,
               [_IDX_1D, _IDX_1D, _ROWS_SCRATCH,
                pltpu.SemaphoreType.DMA, pltpu.SemaphoreType.DMA]),
    }[kind]
    return pl.kernel(
        body,
        out_type=_ROWS_OUT,
        mesh=_sc_mesh(),
        scratch_types=scratch,
    )


def _sc_scatter(table, idx):
    return _sc_move_kernel("scatter")(table, idx)


def _sc_gather(table, idx):
    return _sc_move_kernel("gather")(table, idx)


def _sc_gather_scatter(table, idxg, idxs):
    return _sc_move_kernel("gs")(table, idxg, idxs)


def _bn_affine(st, gb_ref, grow, brow):
    mean = st[0:1, :] * (1.0 / R)
    var = st[1:2, :] * (1.0 / R) - mean * mean
    scale = gb_ref[grow:grow + 1, :] * lax.rsqrt(var + EPS)
    shift = gb_ref[brow:brow + 1, :] - mean * scale
    return scale, shift


def _out_stats(y, i, ost_ref):
    s = jnp.sum(y, axis=0, keepdims=True)
    ss = jnp.sum(y * y, axis=0, keepdims=True)
    blk = jnp.concatenate([s, ss, jnp.zeros((6, D), jnp.float32)], axis=0)

    @pl.when(i == 0)
    def _():
        ost_ref[...] = blk

    @pl.when(i > 0)
    def _():
        ost_ref[...] += blk


_HG = 4
_NG = NH // _HG
_GW = _HG * DH


def _nt(a, b, out_dtype=jnp.float32):
    return lax.dot_general(a, b, (((1,), (1,)), ((), ())),
                           preferred_element_type=out_dtype)


def _block_body(emit_stats, st_ref, gb_ref, x_ref, wqkv_ref, wproj_ref,
                w1_ref, w2_ref, y_ref, ost_ref, h_vmem, st2_vmem,
                wqkv_bf, wproj_bf, w1_bf, w2_bf):
    ph = pl.program_id(0)
    i = pl.program_id(1)

    @pl.when((ph == 0) & (i == 0))
    def _cast_weights():
        wq = wqkv_ref[0]
        rowid = lax.broadcasted_iota(jnp.int32, (3 * D, D), 0)
        wq = jnp.where(rowid < D, wq * (1.0 / math.sqrt(DH)), wq)
        wqkv_bf[...] = wq.astype(jnp.bfloat16)
        wproj_bf[...] = wproj_ref[0].astype(jnp.bfloat16)
        w1_bf[...] = w1_ref[0].astype(jnp.bfloat16)
        w2_bf[...] = w2_ref[0].astype(jnp.bfloat16)

    @pl.when(ph == 0)
    def _attn_phase():
        x = x_ref[...]
        scale, shift = _bn_affine(st_ref[...], gb_ref, 0, 1)
        xn = (x * scale + shift).astype(jnp.bfloat16)
        qkvb = _nt(xn, wqkv_bf[...]).astype(jnp.bfloat16)
        lane = lax.broadcasted_iota(jnp.int32, (1, _GW), 1)
        outs = []
        for g in range(_NG):
            q4 = qkvb[:, g * _GW:(g + 1) * _GW]
            k4 = qkvb[:, D + g * _GW:D + (g + 1) * _GW]
            v4 = qkvb[:, 2 * D + g * _GW:2 * D + (g + 1) * _GW]
            bdk = jnp.concatenate(
                [jnp.where((lane >= h * DH) & (lane < (h + 1) * DH), k4, 0)
                 for h in range(_HG)], axis=0)
            s4 = _nt(q4, bdk)
            e4 = jnp.exp(s4)
            p4 = jnp.concatenate(
                [e4[:, h * WIN:(h + 1) * WIN]
                 / jnp.sum(e4[:, h * WIN:(h + 1) * WIN], axis=-1,
                           keepdims=True) for h in range(_HG)],
                axis=1).astype(jnp.bfloat16)
            bdv = jnp.concatenate(
                [jnp.where((lane >= h * DH) & (lane < (h + 1) * DH), v4, 0)
                 for h in range(_HG)], axis=0)
            outs.append(jnp.dot(p4, bdv, preferred_element_type=jnp.float32))
        o = jnp.concatenate(outs, axis=1).astype(jnp.bfloat16)
        h_out = x + _nt(o, wproj_bf[...])
        h_vmem[pl.ds(i * WIN, WIN), :] = h_out
        s = jnp.sum(h_out, axis=0, keepdims=True)
        ss = jnp.sum(h_out * h_out, axis=0, keepdims=True)
        blk = jnp.concatenate([s, ss, jnp.zeros((6, D), jnp.float32)], axis=0)

        @pl.when(i == 0)
        def _():
            st2_vmem[...] = blk

        @pl.when(i > 0)
        def _():
            st2_vmem[...] += blk

    @pl.when(ph == 1)
    def _mlp_phase():
        hrow = h_vmem[pl.ds(i * WIN, WIN), :]
        scale, shift = _bn_affine(st2_vmem[...], gb_ref, 2, 3)
        hn = (hrow * scale + shift).astype(jnp.bfloat16)
        a = _nt(hn, w1_bf[...])
        a = jnp.maximum(a, 0.0).astype(jnp.bfloat16)
        y = hrow + _nt(a, w2_bf[...])
        y_ref[...] = y
        if emit_stats:
            _out_stats(y, i, ost_ref)


def _block_call(blk_i, st, gb, xp, qkv_w, proj_w, fc1_w, fc2_w,
                emit_stats=True):
    return pl.pallas_call(
        functools.partial(_block_body, emit_stats),
        grid=(2, NWIN),
        in_specs=[
            pl.BlockSpec((8, D), lambda p, i: (0, 0)),
            pl.BlockSpec((8, D), lambda p, i: (0, 0)),
            pl.BlockSpec((WIN, D), lambda p, i: (i * (1 - p), 0)),
            pl.BlockSpec((1, 3 * D, D), lambda p, i, b=blk_i: (b, 0, 0)),
            pl.BlockSpec((1, D, D), lambda p, i, b=blk_i: (b, 0, 0)),
            pl.BlockSpec((1, HID, D), lambda p, i, b=blk_i: (b, 0, 0)),
            pl.BlockSpec((1, D, HID), lambda p, i, b=blk_i: (b, 0, 0)),
        ],
        out_specs=[
            pl.BlockSpec((WIN, D), lambda p, i: (i, 0)),
            pl.BlockSpec((8, D), lambda p, i: (0, 0)),
        ],
        out_shape=[
            jax.ShapeDtypeStruct((R, D), jnp.float32),
            jax.ShapeDtypeStruct((8, D), jnp.float32),
        ],
        scratch_shapes=[
            pltpu.VMEM((R, D), jnp.float32),
            pltpu.VMEM((8, D), jnp.float32),
            pltpu.VMEM((3 * D, D), jnp.bfloat16),
            pltpu.VMEM((D, D), jnp.bfloat16),
            pltpu.VMEM((HID, D), jnp.bfloat16),
            pltpu.VMEM((D, HID), jnp.bfloat16),
        ],
    )(st, gb, xp, qkv_w, proj_w, fc1_w, fc2_w)


_CH = 256
_NCH = N // _CH


_XCH = R // (2 * B)


def _rank_body(prow_ref, x_ref, out_ref, st_ref):
    r = pl.program_id(0)
    xc = x_ref[...]
    s = jnp.sum(xc, axis=0, keepdims=True)
    ss = jnp.sum(xc * xc, axis=0, keepdims=True)
    blk = jnp.concatenate([s, ss, jnp.zeros((6, D), jnp.float32)], axis=0)

    @pl.when(r == 0)
    def _():
        st_ref[...] = blk

    @pl.when(r > 0)
    def _():
        st_ref[...] += blk

    prow = prow_ref[0]
    pr8 = jnp.concatenate(
        [prow[:, c * _CH:(c + 1) * _CH] for c in range(_NCH)], axis=0)
    tcol = jnp.transpose(pr8)
    tri = (lax.broadcasted_iota(jnp.int32, (_CH, _CH), 0)
           < lax.broadcasted_iota(jnp.int32, (_CH, _CH), 1))
    chunks = []
    for ci in range(_NCH):
        pi = prow[:, ci * _CH:(ci + 1) * _CH]
        acc = jnp.zeros((1, _CH), jnp.float32)
        for cj in range(_NCH):
            pj = tcol[:, cj:cj + 1]
            if cj < ci:
                cmp = pj <= pi
            elif cj > ci:
                cmp = pj < pi
            else:
                cmp = (pj < pi) | ((pj == pi) & tri)
            acc = acc + jnp.sum(cmp.astype(jnp.float32), axis=0, keepdims=True)
        chunks.append(acc)
    rank = jnp.concatenate(chunks, axis=1).astype(jnp.int32)
    out_ref[0] = rank + (r % 2) * N


def _rank_call(prow, xf):
    return pl.pallas_call(
        _rank_body,
        grid=(2 * B,),
        in_specs=[
            pl.BlockSpec((1, 1, N), lambda r: (r, 0, 0)),
            pl.BlockSpec((_XCH, D), lambda r: (r, 0)),
        ],
        out_specs=[
            pl.BlockSpec((1, 1, N), lambda r: (r, 0, 0)),
            pl.BlockSpec((8, D), lambda r: (0, 0)),
        ],
        out_shape=[
            jax.ShapeDtypeStruct((2 * B, 1, N), jnp.int32),
            jax.ShapeDtypeStruct((8, D), jnp.float32),
        ],
    )(prow, xf)


def _perm_indices(z, xf):
    kidx = jax.random.key(42)

    def get_proj(key):
        v = jax.random.normal(key, (3,), dtype=jnp.float32)
        v = v / jnp.linalg.norm(v)
        return jnp.einsum('bnc,c->bn', z, v)

    p1 = get_proj(jax.random.fold_in(kidx, 1))
    p2 = get_proj(jax.random.fold_in(kidx, 2))
    P = jnp.concatenate([p1, p2], axis=0)
    ranks3, st = _rank_call(P[:, None, :], xf)
    ranks = ranks3.reshape(2 * B, N)
    rank1_flat = ranks[0:B].reshape(R)
    idx1 = rank1_flat.reshape(SC_NW, ROWS_PER_W)
    idx2 = ranks[B:2 * B].reshape(SC_NW, ROWS_PER_W)
    return idx1, idx2, rank1_flat, st


def kernel(x, z, qkv_w, proj_w, fc1_w, fc2_w, bn1_g, bn1_b, bn2_g, bn2_b):
    xf = x.reshape(R, D)
    idx1, idx2, rank1_flat, st = _perm_indices(z, xf)
    del rank1_flat

    zpad = jnp.zeros((4, D), jnp.float32)
    gb = [jnp.concatenate([bn1_g[i][None], bn1_b[i][None],
                           bn2_g[i][None], bn2_b[i][None], zpad], axis=0)
          for i in range(N_BLOCK)]

    xp = _sc_scatter(xf, idx1)
    y, st3 = _block_call(0, st, gb[0], xp, qkv_w, proj_w, fc1_w, fc2_w)
    xp2 = _sc_gather_scatter(y, idx1, idx2)
    y2, _ = _block_call(1, st3, gb[1], xp2, qkv_w, proj_w, fc1_w, fc2_w,
                        emit_stats=False)
    out = _sc_gather(y2, idx2)
    return out.reshape(B, N, D), z


# speedup vs baseline: 1.1746x; 1.0461x over previous
import functools
import math

import jax
import jax.numpy as jnp
from jax import lax
from jax.experimental import pallas as pl
from jax.experimental.pallas import tpu as pltpu
from jax.experimental.pallas import tpu_sc as plsc

N_BLOCK = 2
WIN = 256
D = 384
NH = 12
DH = D // NH
HID = int(D * 4.0)
B = 2
N = 2048
R = B * N
NWIN = R // WIN
EPS = 1e-5

SC_NC = 2
SC_NS = 16
SC_NW = SC_NC * SC_NS
ROWS_PER_W = R // SC_NW


_NCHK = 4
_CR = ROWS_PER_W // _NCHK


def _sc_scatter_body(table_hbm, idx_hbm, out_hbm, idx_v, rows_v, sem_r):
    wid = lax.axis_index("s") * SC_NC + lax.axis_index("c")
    base = wid * ROWS_PER_W
    pltpu.sync_copy(idx_hbm.at[wid], idx_v)
    pltpu.sync_copy(table_hbm.at[pl.ds(base, ROWS_PER_W)], rows_v)
    pltpu.async_copy(rows_v, out_hbm.at[idx_v], sem_r).wait()


def _sc_gather_body(table_hbm, idx_hbm, out_hbm, idx_v, rows_v, sem_r):
    wid = lax.axis_index("s") * SC_NC + lax.axis_index("c")
    base = wid * ROWS_PER_W
    pltpu.sync_copy(idx_hbm.at[wid], idx_v)
    pltpu.async_copy(table_hbm.at[idx_v], rows_v, sem_r).wait()
    pltpu.sync_copy(rows_v, out_hbm.at[pl.ds(base, ROWS_PER_W)])


def _sc_gs_body(table_hbm, idxg_hbm, idxs_hbm, out_hbm,
                idxg_v, idxs_v, rows_v, sem_g, sem_s):
    wid = lax.axis_index("s") * SC_NC + lax.axis_index("c")
    pltpu.sync_copy(idxg_hbm.at[wid], idxg_v)
    pltpu.sync_copy(idxs_hbm.at[wid], idxs_v)
    pltpu.async_copy(table_hbm.at[idxg_v], rows_v, sem_g).wait()
    pltpu.async_copy(rows_v, out_hbm.at[idxs_v], sem_s).wait()


def _sc_mesh():
    return plsc.VectorSubcoreMesh(
        core_axis_name="c", subcore_axis_name="s",
        num_cores=SC_NC, num_subcores=SC_NS)


_ROWS_SCRATCH = pltpu.VMEM((ROWS_PER_W, D), jnp.float32)
_IDX_1D = pltpu.VMEM((ROWS_PER_W,), jnp.int32)
_ROWS_OUT = jax.ShapeDtypeStruct((R, D), jnp.float32)


@functools.cache
def _sc_move_kernel(kind):
    body, scratch = {
        "scatter": (_sc_scatter_body,
                    [_IDX_1D, _ROWS_SCRATCH, pltpu.SemaphoreType.DMA]),
        "gather": (_sc_gather_body,
                   [_IDX_1D, _ROWS_SCRATCH, pltpu.SemaphoreType.DMA]),
        "gs": (_sc_gs_body,
               [_IDX_1D, _IDX_1D, _ROWS_SCRATCH,
                pltpu.SemaphoreType.DMA, pltpu.SemaphoreType.DMA]),
    }[kind]
    return pl.kernel(
        body,
        out_type=_ROWS_OUT,
        mesh=_sc_mesh(),
        scratch_types=scratch,
    )


def _sc_scatter(table, idx):
    return _sc_move_kernel("scatter")(table, idx)


def _sc_gather(table, idx):
    return _sc_move_kernel("gather")(table, idx)


def _sc_gather_scatter(table, idxg, idxs):
    return _sc_move_kernel("gs")(table, idxg, idxs)


def _bn_affine(st, gb_ref, grow, brow):
    mean = st[0:1, :] * (1.0 / R)
    var = st[1:2, :] * (1.0 / R) - mean * mean
    scale = gb_ref[grow:grow + 1, :] * lax.rsqrt(var + EPS)
    shift = gb_ref[brow:brow + 1, :] - mean * scale
    return scale, shift


def _out_stats(y, i, ost_ref):
    s = jnp.sum(y, axis=0, keepdims=True)
    ss = jnp.sum(y * y, axis=0, keepdims=True)
    blk = jnp.concatenate([s, ss, jnp.zeros((6, D), jnp.float32)], axis=0)

    @pl.when(i == 0)
    def _():
        ost_ref[...] = blk

    @pl.when(i > 0)
    def _():
        ost_ref[...] += blk


_HG = 4
_NG = NH // _HG
_GW = _HG * DH


def _nt(a, b, out_dtype=jnp.float32):
    return lax.dot_general(a, b, (((1,), (1,)), ((), ())),
                           preferred_element_type=out_dtype)


_MCH = 2 * WIN
_NMS = R // _MCH


def _block_body(emit_stats, st_ref, gb_ref, x_ref, wqkv_ref, wproj_ref,
                w1_ref, w2_ref, y_ref, ost_ref, h_vmem, st2_vmem,
                wqkv_bf, wproj_bf, w1_bf, w2_bf):
    i = pl.program_id(0)
    is_attn = i < NWIN

    @pl.when(is_attn & (i == 0))
    def _cast_weights():
        wq = wqkv_ref[0]
        rowid = lax.broadcasted_iota(jnp.int32, (3 * D, D), 0)
        wq = jnp.where(rowid < D, wq * (1.0 / math.sqrt(DH)), wq)
        wqkv_bf[...] = wq.astype(jnp.bfloat16)
        wproj_bf[...] = wproj_ref[0].astype(jnp.bfloat16)
        w1_bf[...] = w1_ref[0].astype(jnp.bfloat16)
        w2_bf[...] = w2_ref[0].astype(jnp.bfloat16)

    @pl.when(is_attn)
    def _attn_phase():
        x = x_ref[...]
        scale, shift = _bn_affine(st_ref[...], gb_ref, 0, 1)
        xn = (x * scale + shift).astype(jnp.bfloat16)
        qkvb = _nt(xn, wqkv_bf[...]).astype(jnp.bfloat16)
        lane = lax.broadcasted_iota(jnp.int32, (1, _GW), 1)
        outs = []
        for g in range(_NG):
            q4 = qkvb[:, g * _GW:(g + 1) * _GW]
            k4 = qkvb[:, D + g * _GW:D + (g + 1) * _GW]
            v4 = qkvb[:, 2 * D + g * _GW:2 * D + (g + 1) * _GW]
            bdk = jnp.concatenate(
                [jnp.where((lane >= h * DH) & (lane < (h + 1) * DH), k4, 0)
                 for h in range(_HG)], axis=0)
            s4 = _nt(q4, bdk)
            e4 = jnp.exp(s4)
            p4 = jnp.concatenate(
                [e4[:, h * WIN:(h + 1) * WIN]
                 / jnp.sum(e4[:, h * WIN:(h + 1) * WIN], axis=-1,
                           keepdims=True) for h in range(_HG)],
                axis=1).astype(jnp.bfloat16)
            bdv = jnp.concatenate(
                [jnp.where((lane >= h * DH) & (lane < (h + 1) * DH), v4, 0)
                 for h in range(_HG)], axis=0)
            outs.append(jnp.dot(p4, bdv, preferred_element_type=jnp.float32))
        o = jnp.concatenate(outs, axis=1).astype(jnp.bfloat16)
        h_out = x + _nt(o, wproj_bf[...])
        h_vmem[pl.ds(i * WIN, WIN), :] = h_out
        s = jnp.sum(h_out, axis=0, keepdims=True)
        ss = jnp.sum(h_out * h_out, axis=0, keepdims=True)
        blk = jnp.concatenate([s, ss, jnp.zeros((6, D), jnp.float32)], axis=0)

        @pl.when(i == 0)
        def _():
            st2_vmem[...] = blk

        @pl.when(i > 0)
        def _():
            st2_vmem[...] += blk

    @pl.when(jnp.logical_not(is_attn))
    def _mlp_phase():
        c = i - NWIN
        hrow = h_vmem[pl.ds(c * _MCH, _MCH), :]
        scale, shift = _bn_affine(st2_vmem[...], gb_ref, 2, 3)
        hn = (hrow * scale + shift).astype(jnp.bfloat16)
        a = _nt(hn, w1_bf[...])
        a = jnp.maximum(a, 0.0).astype(jnp.bfloat16)
        y = hrow + _nt(a, w2_bf[...])
        y_ref[...] = y
        if emit_stats:
            _out_stats(y, c, ost_ref)


def _block_call(blk_i, st, gb, xp, qkv_w, proj_w, fc1_w, fc2_w,
                emit_stats=True):
    return pl.pallas_call(
        functools.partial(_block_body, emit_stats),
        grid=(NWIN + _NMS,),
        in_specs=[
            pl.BlockSpec((8, D), lambda i: (0, 0)),
            pl.BlockSpec((8, D), lambda i: (0, 0)),
            pl.BlockSpec((WIN, D), lambda i: (jnp.where(i < NWIN, i, 0), 0)),
            pl.BlockSpec((1, 3 * D, D), lambda i, b=blk_i: (b, 0, 0)),
            pl.BlockSpec((1, D, D), lambda i, b=blk_i: (b, 0, 0)),
            pl.BlockSpec((1, HID, D), lambda i, b=blk_i: (b, 0, 0)),
            pl.BlockSpec((1, D, HID), lambda i, b=blk_i: (b, 0, 0)),
        ],
        out_specs=[
            pl.BlockSpec((_MCH, D),
                         lambda i: (jnp.where(i < NWIN, i % _NMS,
                                              i - NWIN), 0)),
            pl.BlockSpec((8, D), lambda i: (0, 0)),
        ],
        out_shape=[
            jax.ShapeDtypeStruct((R, D), jnp.float32),
            jax.ShapeDtypeStruct((8, D), jnp.float32),
        ],
        scratch_shapes=[
            pltpu.VMEM((R, D), jnp.float32),
            pltpu.VMEM((8, D), jnp.float32),
            pltpu.VMEM((3 * D, D), jnp.bfloat16),
            pltpu.VMEM((D, D), jnp.bfloat16),
            pltpu.VMEM((HID, D), jnp.bfloat16),
            pltpu.VMEM((D, HID), jnp.bfloat16),
        ],
    )(st, gb, xp, qkv_w, proj_w, fc1_w, fc2_w)


_CH = 256
_NCH = N // _CH


_XCH = R // (2 * B)


def _rank_body(prow_ref, x_ref, out_ref, st_ref):
    r = pl.program_id(0)
    xc = x_ref[...]
    s = jnp.sum(xc, axis=0, keepdims=True)
    ss = jnp.sum(xc * xc, axis=0, keepdims=True)
    blk = jnp.concatenate([s, ss, jnp.zeros((6, D), jnp.float32)], axis=0)

    @pl.when(r == 0)
    def _():
        st_ref[...] = blk

    @pl.when(r > 0)
    def _():
        st_ref[...] += blk

    prow = prow_ref[0]
    pr8 = jnp.concatenate(
        [prow[:, c * _CH:(c + 1) * _CH] for c in range(_NCH)], axis=0)
    tcol = jnp.transpose(pr8)
    tri = (lax.broadcasted_iota(jnp.int32, (_CH, _CH), 0)
           < lax.broadcasted_iota(jnp.int32, (_CH, _CH), 1))
    chunks = []
    for ci in range(_NCH):
        pi = prow[:, ci * _CH:(ci + 1) * _CH]
        acc = jnp.zeros((1, _CH), jnp.float32)
        for cj in range(_NCH):
            pj = tcol[:, cj:cj + 1]
            if cj < ci:
                cmp = pj <= pi
            elif cj > ci:
                cmp = pj < pi
            else:
                cmp = (pj < pi) | ((pj == pi) & tri)
            acc = acc + jnp.sum(cmp.astype(jnp.float32), axis=0, keepdims=True)
        chunks.append(acc)
    rank = jnp.concatenate(chunks, axis=1).astype(jnp.int32)
    out_ref[0] = rank + (r % 2) * N


def _rank_call(prow, xf):
    return pl.pallas_call(
        _rank_body,
        grid=(2 * B,),
        in_specs=[
            pl.BlockSpec((1, 1, N), lambda r: (r, 0, 0)),
            pl.BlockSpec((_XCH, D), lambda r: (r, 0)),
        ],
        out_specs=[
            pl.BlockSpec((1, 1, N), lambda r: (r, 0, 0)),
            pl.BlockSpec((8, D), lambda r: (0, 0)),
        ],
        out_shape=[
            jax.ShapeDtypeStruct((2 * B, 1, N), jnp.int32),
            jax.ShapeDtypeStruct((8, D), jnp.float32),
        ],
    )(prow, xf)


def _perm_indices(z, xf):
    kidx = jax.random.key(42)

    def get_proj(key):
        v = jax.random.normal(key, (3,), dtype=jnp.float32)
        v = v / jnp.linalg.norm(v)
        return jnp.einsum('bnc,c->bn', z, v)

    p1 = get_proj(jax.random.fold_in(kidx, 1))
    p2 = get_proj(jax.random.fold_in(kidx, 2))
    P = jnp.concatenate([p1, p2], axis=0)
    ranks3, st = _rank_call(P[:, None, :], xf)
    ranks = ranks3.reshape(2 * B, N)
    rank1_flat = ranks[0:B].reshape(R)
    idx1 = rank1_flat.reshape(SC_NW, ROWS_PER_W)
    idx2 = ranks[B:2 * B].reshape(SC_NW, ROWS_PER_W)
    return idx1, idx2, rank1_flat, st


def kernel(x, z, qkv_w, proj_w, fc1_w, fc2_w, bn1_g, bn1_b, bn2_g, bn2_b):
    xf = x.reshape(R, D)
    idx1, idx2, rank1_flat, st = _perm_indices(z, xf)
    del rank1_flat

    zpad = jnp.zeros((4, D), jnp.float32)
    gb = [jnp.concatenate([bn1_g[i][None], bn1_b[i][None],
                           bn2_g[i][None], bn2_b[i][None], zpad], axis=0)
          for i in range(N_BLOCK)]

    xp = _sc_scatter(xf, idx1)
    y, st3 = _block_call(0, st, gb[0], xp, qkv_w, proj_w, fc1_w, fc2_w)
    xp2 = _sc_gather_scatter(y, idx1, idx2)
    y2, _ = _block_call(1, st3, gb[1], xp2, qkv_w, proj_w, fc1_w, fc2_w,
                        emit_stats=False)
    out = _sc_gather(y2, idx2)
    return out.reshape(B, N, D), z


# speedup vs baseline: 1.2303x; 1.0474x over previous
import functools
import math

import jax
import jax.numpy as jnp
from jax import lax
from jax.experimental import pallas as pl
from jax.experimental.pallas import tpu as pltpu
from jax.experimental.pallas import tpu_sc as plsc

N_BLOCK = 2
WIN = 256
D = 384
NH = 12
DH = D // NH
HID = int(D * 4.0)
B = 2
N = 2048
R = B * N
NWIN = R // WIN
EPS = 1e-5

SC_NC = 2
SC_NS = 16
SC_NW = SC_NC * SC_NS
ROWS_PER_W = R // SC_NW


_NCHK = 4
_CR = ROWS_PER_W // _NCHK


def _sc_scatter_body(table_hbm, idx_hbm, out_hbm, idx_v, rows_v, sem_r):
    wid = lax.axis_index("s") * SC_NC + lax.axis_index("c")
    base = wid * ROWS_PER_W
    pltpu.sync_copy(idx_hbm.at[wid], idx_v)
    pltpu.sync_copy(table_hbm.at[pl.ds(base, ROWS_PER_W)], rows_v)
    pltpu.async_copy(rows_v, out_hbm.at[idx_v], sem_r).wait()


def _sc_gather_body(table_hbm, idx_hbm, out_hbm, idx_v, rows_v, sem_r):
    wid = lax.axis_index("s") * SC_NC + lax.axis_index("c")
    base = wid * ROWS_PER_W
    pltpu.sync_copy(idx_hbm.at[wid], idx_v)
    pltpu.async_copy(table_hbm.at[idx_v], rows_v, sem_r).wait()
    pltpu.sync_copy(rows_v, out_hbm.at[pl.ds(base, ROWS_PER_W)])


def _sc_gs_body(table_hbm, idxg_hbm, idxs_hbm, out_hbm,
                idxg_v, idxs_v, rows_v, sem_g, sem_s):
    wid = lax.axis_index("s") * SC_NC + lax.axis_index("c")
    pltpu.sync_copy(idxg_hbm.at[wid], idxg_v)
    pltpu.sync_copy(idxs_hbm.at[wid], idxs_v)
    pltpu.async_copy(table_hbm.at[idxg_v], rows_v, sem_g).wait()
    pltpu.async_copy(rows_v, out_hbm.at[idxs_v], sem_s).wait()


def _sc_mesh():
    return plsc.VectorSubcoreMesh(
        core_axis_name="c", subcore_axis_name="s",
        num_cores=SC_NC, num_subcores=SC_NS)


_ROWS_SCRATCH = pltpu.VMEM((ROWS_PER_W, D), jnp.float32)
_IDX_1D = pltpu.VMEM((ROWS_PER_W,), jnp.int32)
_ROWS_OUT = jax.ShapeDtypeStruct((R, D), jnp.float32)


@functools.cache
def _sc_move_kernel(kind):
    body, scratch = {
        "scatter": (_sc_scatter_body,
                    [_IDX_1D, _ROWS_SCRATCH, pltpu.SemaphoreType.DMA]),
        "gather": (_sc_gather_body,
                   [_IDX_1D, _ROWS_SCRATCH, pltpu.SemaphoreType.DMA]),
        "gs": (_sc_gs_body,
               [_IDX_1D, _IDX_1D, _ROWS_SCRATCH,
                pltpu.SemaphoreType.DMA, pltpu.SemaphoreType.DMA]),
    }[kind]
    return pl.kernel(
        body,
        out_type=_ROWS_OUT,
        mesh=_sc_mesh(),
        scratch_types=scratch,
    )


def _sc_scatter(table, idx):
    return _sc_move_kernel("scatter")(table, idx)


def _sc_gather(table, idx):
    return _sc_move_kernel("gather")(table, idx)


def _sc_gather_scatter(table, idxg, idxs):
    return _sc_move_kernel("gs")(table, idxg, idxs)


def _bn_affine(st, gb_ref, grow, brow):
    mean = st[0:1, :] * (1.0 / R)
    var = st[1:2, :] * (1.0 / R) - mean * mean
    scale = gb_ref[grow:grow + 1, :] * lax.rsqrt(var + EPS)
    shift = gb_ref[brow:brow + 1, :] - mean * scale
    return scale, shift


def _out_stats(y, i, ost_ref):
    s = jnp.sum(y, axis=0, keepdims=True)
    ss = jnp.sum(y * y, axis=0, keepdims=True)
    blk = jnp.concatenate([s, ss, jnp.zeros((6, D), jnp.float32)], axis=0)

    @pl.when(i == 0)
    def _():
        ost_ref[...] = blk

    @pl.when(i > 0)
    def _():
        ost_ref[...] += blk


_HG = 4
_NG = NH // _HG
_GW = _HG * DH


def _nt(a, b, out_dtype=jnp.float32):
    return lax.dot_general(a, b, (((1,), (1,)), ((), ())),
                           preferred_element_type=out_dtype)


_MCH = 2 * WIN
_NMS = R // _MCH


def _block_body(emit_stats, st_ref, gb_ref, x_ref, wqkv_ref, wproj_ref,
                w1_ref, w2_ref, y_ref, ost_ref, h_vmem, st2_vmem,
                wqkv_bf, wproj_bf, w1_bf, w2_bf):
    i = pl.program_id(0)
    is_attn = i < _NMS

    @pl.when(is_attn & (i == 0))
    def _cast_weights():
        wq = wqkv_ref[0]
        rowid = lax.broadcasted_iota(jnp.int32, (3 * D, D), 0)
        wq = jnp.where(rowid < D, wq * (1.0 / math.sqrt(DH)), wq)
        wqkv_bf[...] = wq.astype(jnp.bfloat16)
        wproj_bf[...] = wproj_ref[0].astype(jnp.bfloat16)
        w1_bf[...] = w1_ref[0].astype(jnp.bfloat16)
        w2_bf[...] = w2_ref[0].astype(jnp.bfloat16)

    @pl.when(is_attn)
    def _attn_phase():
        x2 = x_ref[...]
        scale, shift = _bn_affine(st_ref[...], gb_ref, 0, 1)
        lane = lax.broadcasted_iota(jnp.int32, (1, _GW), 1)
        blk = jnp.zeros((8, D), jnp.float32)
        for w in range(2):
            x = x2[w * WIN:(w + 1) * WIN, :]
            xn = (x * scale + shift).astype(jnp.bfloat16)
            qkvb = _nt(xn, wqkv_bf[...]).astype(jnp.bfloat16)
            outs = []
            for g in range(_NG):
                q4 = qkvb[:, g * _GW:(g + 1) * _GW]
                k4 = qkvb[:, D + g * _GW:D + (g + 1) * _GW]
                v4 = qkvb[:, 2 * D + g * _GW:2 * D + (g + 1) * _GW]
                bdk = jnp.concatenate(
                    [jnp.where((lane >= h * DH) & (lane < (h + 1) * DH),
                               k4, 0) for h in range(_HG)], axis=0)
                s4 = _nt(q4, bdk)
                e4 = jnp.exp(s4)
                p4 = jnp.concatenate(
                    [e4[:, h * WIN:(h + 1) * WIN]
                     / jnp.sum(e4[:, h * WIN:(h + 1) * WIN], axis=-1,
                               keepdims=True) for h in range(_HG)],
                    axis=1).astype(jnp.bfloat16)
                bdv = jnp.concatenate(
                    [jnp.where((lane >= h * DH) & (lane < (h + 1) * DH),
                               v4, 0) for h in range(_HG)], axis=0)
                outs.append(jnp.dot(p4, bdv,
                                    preferred_element_type=jnp.float32))
            o = jnp.concatenate(outs, axis=1).astype(jnp.bfloat16)
            h_out = x + _nt(o, wproj_bf[...])
            h_vmem[pl.ds((2 * i + w) * WIN, WIN), :] = h_out
            s = jnp.sum(h_out, axis=0, keepdims=True)
            ss = jnp.sum(h_out * h_out, axis=0, keepdims=True)
            blk = blk + jnp.concatenate(
                [s, ss, jnp.zeros((6, D), jnp.float32)], axis=0)

        @pl.when(i == 0)
        def _():
            st2_vmem[...] = blk

        @pl.when(i > 0)
        def _():
            st2_vmem[...] += blk

    @pl.when(jnp.logical_not(is_attn))
    def _mlp_phase():
        c = i - _NMS
        hrow = h_vmem[pl.ds(c * _MCH, _MCH), :]
        scale, shift = _bn_affine(st2_vmem[...], gb_ref, 2, 3)
        hn = (hrow * scale + shift).astype(jnp.bfloat16)
        a = _nt(hn, w1_bf[...])
        a = jnp.maximum(a, 0.0).astype(jnp.bfloat16)
        y = hrow + _nt(a, w2_bf[...])
        y_ref[...] = y
        if emit_stats:
            _out_stats(y, c, ost_ref)


def _block_call(blk_i, st, gb, xp, qkv_w, proj_w, fc1_w, fc2_w,
                emit_stats=True):
    return pl.pallas_call(
        functools.partial(_block_body, emit_stats),
        grid=(2 * _NMS,),
        in_specs=[
            pl.BlockSpec((8, D), lambda i: (0, 0)),
            pl.BlockSpec((8, D), lambda i: (0, 0)),
            pl.BlockSpec((_MCH, D), lambda i: (jnp.where(i < _NMS, i, 0), 0)),
            pl.BlockSpec((1, 3 * D, D), lambda i, b=blk_i: (b, 0, 0)),
            pl.BlockSpec((1, D, D), lambda i, b=blk_i: (b, 0, 0)),
            pl.BlockSpec((1, HID, D), lambda i, b=blk_i: (b, 0, 0)),
            pl.BlockSpec((1, D, HID), lambda i, b=blk_i: (b, 0, 0)),
        ],
        out_specs=[
            pl.BlockSpec((_MCH, D),
                         lambda i: (jnp.where(i < _NMS, i, i - _NMS), 0)),
            pl.BlockSpec((8, D), lambda i: (0, 0)),
        ],
        out_shape=[
            jax.ShapeDtypeStruct((R, D), jnp.float32),
            jax.ShapeDtypeStruct((8, D), jnp.float32),
        ],
        scratch_shapes=[
            pltpu.VMEM((R, D), jnp.float32),
            pltpu.VMEM((8, D), jnp.float32),
            pltpu.VMEM((3 * D, D), jnp.bfloat16),
            pltpu.VMEM((D, D), jnp.bfloat16),
            pltpu.VMEM((HID, D), jnp.bfloat16),
            pltpu.VMEM((D, HID), jnp.bfloat16),
        ],
    )(st, gb, xp, qkv_w, proj_w, fc1_w, fc2_w)


_CH = 256
_NCH = N // _CH


_XCH = R // (2 * B)


def _rank_body(prow_ref, x_ref, out_ref, st_ref):
    r = pl.program_id(0)
    xc = x_ref[...]
    s = jnp.sum(xc, axis=0, keepdims=True)
    ss = jnp.sum(xc * xc, axis=0, keepdims=True)
    blk = jnp.concatenate([s, ss, jnp.zeros((6, D), jnp.float32)], axis=0)

    @pl.when(r == 0)
    def _():
        st_ref[...] = blk

    @pl.when(r > 0)
    def _():
        st_ref[...] += blk

    prow = prow_ref[0]
    pr8 = jnp.concatenate(
        [prow[:, c * _CH:(c + 1) * _CH] for c in range(_NCH)], axis=0)
    tcol = jnp.transpose(pr8)
    tri = (lax.broadcasted_iota(jnp.int32, (_CH, _CH), 0)
           < lax.broadcasted_iota(jnp.int32, (_CH, _CH), 1))
    chunks = []
    for ci in range(_NCH):
        pi = prow[:, ci * _CH:(ci + 1) * _CH]
        acc = jnp.zeros((1, _CH), jnp.float32)
        for cj in range(_NCH):
            pj = tcol[:, cj:cj + 1]
            if cj < ci:
                cmp = pj <= pi
            elif cj > ci:
                cmp = pj < pi
            else:
                cmp = (pj < pi) | ((pj == pi) & tri)
            acc = acc + jnp.sum(cmp.astype(jnp.float32), axis=0, keepdims=True)
        chunks.append(acc)
    rank = jnp.concatenate(chunks, axis=1).astype(jnp.int32)
    out_ref[0] = rank + (r % 2) * N


def _rank_call(prow, xf):
    return pl.pallas_call(
        _rank_body,
        grid=(2 * B,),
        in_specs=[
            pl.BlockSpec((1, 1, N), lambda r: (r, 0, 0)),
            pl.BlockSpec((_XCH, D), lambda r: (r, 0)),
        ],
        out_specs=[
            pl.BlockSpec((1, 1, N), lambda r: (r, 0, 0)),
            pl.BlockSpec((8, D), lambda r: (0, 0)),
        ],
        out_shape=[
            jax.ShapeDtypeStruct((2 * B, 1, N), jnp.int32),
            jax.ShapeDtypeStruct((8, D), jnp.float32),
        ],
    )(prow, xf)


def _perm_indices(z, xf):
    kidx = jax.random.key(42)

    def get_proj(key):
        v = jax.random.normal(key, (3,), dtype=jnp.float32)
        v = v / jnp.linalg.norm(v)
        return jnp.einsum('bnc,c->bn', z, v)

    p1 = get_proj(jax.random.fold_in(kidx, 1))
    p2 = get_proj(jax.random.fold_in(kidx, 2))
    P = jnp.concatenate([p1, p2], axis=0)
    ranks3, st = _rank_call(P[:, None, :], xf)
    ranks = ranks3.reshape(2 * B, N)
    rank1_flat = ranks[0:B].reshape(R)
    idx1 = rank1_flat.reshape(SC_NW, ROWS_PER_W)
    idx2 = ranks[B:2 * B].reshape(SC_NW, ROWS_PER_W)
    return idx1, idx2, rank1_flat, st


def kernel(x, z, qkv_w, proj_w, fc1_w, fc2_w, bn1_g, bn1_b, bn2_g, bn2_b):
    xf = x.reshape(R, D)
    idx1, idx2, rank1_flat, st = _perm_indices(z, xf)
    del rank1_flat

    zpad = jnp.zeros((4, D), jnp.float32)
    gb = [jnp.concatenate([bn1_g[i][None], bn1_b[i][None],
                           bn2_g[i][None], bn2_b[i][None], zpad], axis=0)
          for i in range(N_BLOCK)]

    xp = _sc_scatter(xf, idx1)
    y, st3 = _block_call(0, st, gb[0], xp, qkv_w, proj_w, fc1_w, fc2_w)
    xp2 = _sc_gather_scatter(y, idx1, idx2)
    y2, _ = _block_call(1, st3, gb[1], xp2, qkv_w, proj_w, fc1_w, fc2_w,
                        emit_stats=False)
    out = _sc_gather(y2, idx2)
    return out.reshape(B, N, D), z
